# baseline (speedup 1.0000x reference)
import functools
import math

import jax
import jax.numpy as jnp
from jax.experimental import pallas as pl
from jax.experimental.pallas import tpu as pltpu

K = 2


def _router_kernel(x_ref, w_ref, bsw_ref, idx_ref, gval_ref, acc_ref):
    i = pl.program_id(0)

    @pl.when(i == 0)
    def _init():
        acc_ref[...] = jnp.zeros_like(acc_ref)

    xb = x_ref[...]
    xb2 = xb.reshape(xb.shape[0], xb.shape[1] * xb.shape[2])
    acc_ref[...] += jax.lax.dot_general(
        xb2, w_ref[...],
        (((1,), (1,)), ((), ())),
        preferred_element_type=jnp.float32,
    )

    @pl.when(i == pl.num_programs(0) - 1)
    def _finish():
        logits = acc_ref[...] + bsw_ref[...]
        e = logits.shape[1]
        m = jnp.max(logits, axis=1, keepdims=True)
        p = jnp.exp(logits - m)
        p = p / jnp.sum(p, axis=1, keepdims=True)
        col = jax.lax.broadcasted_iota(jnp.int32, p.shape, 1)
        m1 = jnp.max(p, axis=1, keepdims=True)
        i1 = jnp.min(jnp.where(p == m1, col, e), axis=1, keepdims=True)
        p2 = jnp.where(col == i1, -1.0, p)
        m2 = jnp.max(p2, axis=1, keepdims=True)
        i2 = jnp.min(jnp.where(p2 == m2, col, e), axis=1, keepdims=True)
        idx_ref[...] = jnp.concatenate([i1, i2], axis=1)
        gval_ref[...] = jnp.concatenate([m1, m2], axis=1)


def _expert_kernel(idx_s, gval_s, x_ref, w1_ref, b1_ref, w2_ref, b2_ref,
                   o_ref):
    del idx_s
    b = pl.program_id(0)
    k = pl.program_id(2)
    g = gval_s[b, k]
    xb = (x_ref[0] * g).astype(jnp.bfloat16)
    h = jax.lax.dot_general(
        xb, w1_ref[0].astype(jnp.bfloat16), (((1,), (1,)), ((), ())),
        preferred_element_type=jnp.float32,
    ) + b1_ref[0]
    a = 0.5 * h * (1.0 + jax.lax.erf(h * (1.0 / math.sqrt(2.0))))
    y = jax.lax.dot_general(
        a.astype(jnp.bfloat16), w2_ref[...].astype(jnp.bfloat16),
        (((1,), (1,)), ((), ())),
        preferred_element_type=jnp.float32,
    )

    @pl.when(k == 0)
    def _first():
        o_ref[0] = y + b2_ref[...]

    @pl.when(k > 0)
    def _rest():
        o_ref[0] += y


@jax.jit
def kernel(x, Wsw, bsw, W1, b1, W2, b2):
    B, S, D = x.shape
    E, SH, SD = W1.shape
    H = W2.shape[1]

    NC = 16
    Ss = S // NC
    C = Ss * D
    idx, gval = pl.pallas_call(
        _router_kernel,
        grid=(NC,),
        in_specs=[
            pl.BlockSpec((B, Ss, D), lambda i: (0, i, 0)),
            pl.BlockSpec((E, C), lambda i: (0, i)),
            pl.BlockSpec((1, E), lambda i: (0, 0)),
        ],
        out_specs=[
            pl.BlockSpec((B, K), lambda i: (0, 0)),
            pl.BlockSpec((B, K), lambda i: (0, 0)),
        ],
        out_shape=[
            jax.ShapeDtypeStruct((B, K), jnp.int32),
            jax.ShapeDtypeStruct((B, K), jnp.float32),
        ],
        scratch_shapes=[pltpu.VMEM((B, E), jnp.float32)],
        compiler_params=pltpu.CompilerParams(
            dimension_semantics=("arbitrary",),
        ),
    )(x, Wsw, bsw.reshape(1, E))

    St = 1024
    S_TILES = S // St
    grid_spec = pltpu.PrefetchScalarGridSpec(
        num_scalar_prefetch=2,
        grid=(B, S_TILES, K),
        in_specs=[
            pl.BlockSpec((1, St, SD), lambda b, s, k, idx_s, gv: (b, s, idx_s[b, k])),
            pl.BlockSpec((1, SH, SD), lambda b, s, k, idx_s, gv: (idx_s[b, k], 0, 0)),
            pl.BlockSpec((1, 1, SH), lambda b, s, k, idx_s, gv: (idx_s[b, k], 0, 0)),
            pl.BlockSpec((D, SH), lambda b, s, k, idx_s, gv: (0, idx_s[b, k])),
            pl.BlockSpec((1, D), lambda b, s, k, idx_s, gv: (0, 0)),
        ],
        out_specs=pl.BlockSpec((1, St, D), lambda b, s, k, idx_s, gv: (b, s, 0)),
    )
    y = pl.pallas_call(
        _expert_kernel,
        grid_spec=grid_spec,
        out_shape=jax.ShapeDtypeStruct((B, S, D), jnp.float32),
        compiler_params=pltpu.CompilerParams(
            dimension_semantics=("parallel", "parallel", "arbitrary"),
        ),
    )(idx, gval, x, W1, b1.reshape(E, 1, SH), W2, b2.reshape(1, D))
    return y


# speedup vs baseline: 1.0455x; 1.0455x over previous
import functools
import math

import jax
import jax.numpy as jnp
from jax.experimental import pallas as pl
from jax.experimental.pallas import tpu as pltpu

K = 2


def _router_kernel(x_ref, w_ref, bsw_ref, idx_ref, gval_ref, acc_ref):
    i = pl.program_id(0)

    @pl.when(i == 0)
    def _init():
        acc_ref[...] = jnp.zeros_like(acc_ref)

    xb = x_ref[...]
    xb2 = xb.reshape(xb.shape[0], xb.shape[1] * xb.shape[2])
    acc_ref[...] += jax.lax.dot_general(
        xb2, w_ref[...],
        (((1,), (1,)), ((), ())),
        preferred_element_type=jnp.float32,
    )

    @pl.when(i == pl.num_programs(0) - 1)
    def _finish():
        logits = acc_ref[...] + bsw_ref[...]
        e = logits.shape[1]
        m = jnp.max(logits, axis=1, keepdims=True)
        p = jnp.exp(logits - m)
        p = p / jnp.sum(p, axis=1, keepdims=True)
        col = jax.lax.broadcasted_iota(jnp.int32, p.shape, 1)
        m1 = jnp.max(p, axis=1, keepdims=True)
        i1 = jnp.min(jnp.where(p == m1, col, e), axis=1, keepdims=True)
        p2 = jnp.where(col == i1, -1.0, p)
        m2 = jnp.max(p2, axis=1, keepdims=True)
        i2 = jnp.min(jnp.where(p2 == m2, col, e), axis=1, keepdims=True)
        idx_ref[...] = jnp.concatenate([i1, i2], axis=1)
        gval_ref[...] = jnp.concatenate([m1, m2], axis=1)


def _one_expert(x_ref, w1_ref, b1_ref, w2_ref, g):
    xb = x_ref[0] * g
    h = jax.lax.dot_general(
        xb, w1_ref[0], (((1,), (1,)), ((), ())),
        preferred_element_type=jnp.float32,
    ) + b1_ref[0]
    a = 0.5 * h * (1.0 + jax.lax.erf(h * (1.0 / math.sqrt(2.0))))
    return jax.lax.dot_general(
        a, w2_ref[...], (((1,), (1,)), ((), ())),
        preferred_element_type=jnp.float32,
    )


def _expert_kernel(idx_s, gval_s, x0_ref, x1_ref, w1a_ref, w1b_ref,
                   b1a_ref, b1b_ref, w2a_ref, w2b_ref, b2_ref, o_ref):
    del idx_s
    b = pl.program_id(0)
    y0 = _one_expert(x0_ref, w1a_ref, b1a_ref, w2a_ref, gval_s[b, 0])
    y1 = _one_expert(x1_ref, w1b_ref, b1b_ref, w2b_ref, gval_s[b, 1])
    o_ref[0] = (y0 + y1) + b2_ref[...]


@jax.jit
def kernel(x, Wsw, bsw, W1, b1, W2, b2):
    B, S, D = x.shape
    E, SH, SD = W1.shape
    H = W2.shape[1]

    NC = 16
    Ss = S // NC
    C = Ss * D
    idx, gval = pl.pallas_call(
        _router_kernel,
        grid=(NC,),
        in_specs=[
            pl.BlockSpec((B, Ss, D), lambda i: (0, i, 0)),
            pl.BlockSpec((E, C), lambda i: (0, i)),
            pl.BlockSpec((1, E), lambda i: (0, 0)),
        ],
        out_specs=[
            pl.BlockSpec((B, K), lambda i: (0, 0)),
            pl.BlockSpec((B, K), lambda i: (0, 0)),
        ],
        out_shape=[
            jax.ShapeDtypeStruct((B, K), jnp.int32),
            jax.ShapeDtypeStruct((B, K), jnp.float32),
        ],
        scratch_shapes=[pltpu.VMEM((B, E), jnp.float32)],
        compiler_params=pltpu.CompilerParams(
            dimension_semantics=("arbitrary",),
        ),
    )(x, Wsw, bsw.reshape(1, E))

    St = 512
    S_TILES = S // St
    b1r = b1.reshape(E, 1, SH)
    b2r = b2.reshape(1, D)
    grid_spec = pltpu.PrefetchScalarGridSpec(
        num_scalar_prefetch=2,
        grid=(B, S_TILES),
        in_specs=[
            pl.BlockSpec((1, St, SD), lambda b, s, idx_s, gv: (b, s, idx_s[b, 0])),
            pl.BlockSpec((1, St, SD), lambda b, s, idx_s, gv: (b, s, idx_s[b, 1])),
            pl.BlockSpec((1, SH, SD), lambda b, s, idx_s, gv: (idx_s[b, 0], 0, 0)),
            pl.BlockSpec((1, SH, SD), lambda b, s, idx_s, gv: (idx_s[b, 1], 0, 0)),
            pl.BlockSpec((1, 1, SH), lambda b, s, idx_s, gv: (idx_s[b, 0], 0, 0)),
            pl.BlockSpec((1, 1, SH), lambda b, s, idx_s, gv: (idx_s[b, 1], 0, 0)),
            pl.BlockSpec((D, SH), lambda b, s, idx_s, gv: (0, idx_s[b, 0])),
            pl.BlockSpec((D, SH), lambda b, s, idx_s, gv: (0, idx_s[b, 1])),
            pl.BlockSpec((1, D), lambda b, s, idx_s, gv: (0, 0)),
        ],
        out_specs=pl.BlockSpec((1, St, D), lambda b, s, idx_s, gv: (b, s, 0)),
    )
    y = pl.pallas_call(
        _expert_kernel,
        grid_spec=grid_spec,
        out_shape=jax.ShapeDtypeStruct((B, S, D), jnp.float32),
        compiler_params=pltpu.CompilerParams(
            dimension_semantics=("parallel", "parallel"),
        ),
    )(idx, gval, x, x, W1, W1, b1r, b1r, W2, W2, b2r)
    return y
